# Initial kernel scaffold; baseline (speedup 1.0000x reference)
import jax
import jax.numpy as jnp
from jax.experimental import pallas as pl


def kernel(x, edge_index, W, a_src, a_dst):
    raise NotImplementedError("write your pallas kernel here")


# speedup vs baseline: 42.9272x; 42.9272x over previous
import functools

import jax
import jax.numpy as jnp
from jax import lax
from jax.experimental import pallas as pl
from jax.experimental.pallas import tpu as pltpu
from jax.experimental.pallas import tpu_sc as plsc

N = 10000
E = 320000
IN_F = 128
OUT_F = 16
H = 8
FC = H * OUT_F

NC = 2
NS = 16
NW = NC * NS

NPAD = 10240
NPT = NPAD // NS

EPAD = NW * NPAD
EPT = EPAD // NW
BW = 128
NB = EPT // BW
CROWS = 8
CH = NB // CROWS
ECH = CROWS * BW

ROWB = 1024
GRID1 = NPAD // ROWB


def _tc_proj_body(x_ref, wcat_ref, ams_ref, amd_ref, h_ref, as_ref, ad_ref):
    xb = x_ref[...]
    hb = jnp.dot(xb, wcat_ref[...], preferred_element_type=jnp.float32)
    h_ref[...] = hb
    dn = (((0,), (1,)), ((), ()))
    as_ref[...] = lax.dot_general(ams_ref[...], hb, dn,
                                  preferred_element_type=jnp.float32)
    ad_ref[...] = lax.dot_general(amd_ref[...], hb, dn,
                                  preferred_element_type=jnp.float32)


def _tc_final_body(d_ref, sel_ref, p0_ref, p1_ref, x_ref, o_ref):
    den8 = jnp.sum(d_ref[...], axis=0)
    den128 = lax.dot_general(den8, sel_ref[...],
                             (((0,), (0,)), ((), ())),
                             preferred_element_type=jnp.float32)
    recip = 1.0 / (den128 + 1e-10)
    o_ref[...] = (p0_ref[...] + p1_ref[...]) * recip + x_ref[...]


_MESH = plsc.VectorSubcoreMesh(core_axis_name="c", subcore_axis_name="s")


@functools.partial(
    pl.kernel,
    out_type=[
        jax.ShapeDtypeStruct((NW, H, BW // 16 * 10, 128), jnp.float32),
        jax.ShapeDtypeStruct((NW * H * EPT,), jnp.float32),
    ],
    mesh=_MESH,
    compiler_params=pltpu.CompilerParams(needs_layout_passes=False),
    scratch_types=[
        pltpu.VMEM((EPT,), jnp.int32),
        pltpu.VMEM((EPT,), jnp.int32),
        pltpu.VMEM((NPAD,), jnp.float32),
        pltpu.VMEM((NPAD,), jnp.float32),
        pltpu.VMEM((NPAD // 128, 128), jnp.float32),
        pltpu.VMEM((EPT,), jnp.float32),
    ],
)
def _sc_logits(alphas_hbm, alphad_hbm, srcf_hbm, dstf_hbm,
               dpart_hbm, wall_hbm,
               srcf_v, dstf_v, as_v, ad_v, den_v, w_v):
    c = lax.axis_index("c")
    s = lax.axis_index("s")
    wid = c * NS + s

    pltpu.sync_copy(srcf_hbm.at[pl.ds(wid * EPT, EPT)], srcf_v)
    pltpu.sync_copy(dstf_hbm.at[pl.ds(wid * EPT, EPT)], dstf_v)

    def head(h, carry):
        pltpu.sync_copy(alphas_hbm.at[pl.ds(h * NPAD, NPAD)], as_v)
        pltpu.sync_copy(alphad_hbm.at[pl.ds(h * NPAD, NPAD)], ad_v)

        def dz(r, carry2):
            def dzc(k, carry3):
                den_v[r, pl.ds(k * 16, 16)] = jnp.zeros((16,), jnp.float32)
                return carry3
            lax.fori_loop(0, 128 // 16, dzc, 0)
            return carry2
        lax.fori_loop(0, NPAD // 128, dz, 0)

        def grp(i, carry2):
            off = i * 16
            s16 = srcf_v[pl.ds(off, 16)]
            d16 = dstf_v[pl.ds(off, 16)]
            a1 = plsc.load_gather(as_v, [s16])
            a2 = plsc.load_gather(ad_v, [d16])
            e = a1 + a2
            e = jnp.where(e > 0.0, e, 0.2 * e)
            w = jnp.exp(e)
            w_v[pl.ds(off, 16)] = w
            plsc.addupdate_scatter(
                den_v, [jnp.right_shift(d16, 7),
                        jnp.bitwise_and(d16, 127)], w)
            return carry2
        lax.fori_loop(0, EPT // 16, grp, 0)

        pltpu.sync_copy(den_v, dpart_hbm.at[wid, h])
        pltpu.sync_copy(w_v, wall_hbm.at[pl.ds((wid * H + h) * EPT, EPT)])
        return carry
    lax.fori_loop(0, H, head, 0)


@functools.partial(
    pl.kernel,
    out_type=jax.ShapeDtypeStruct((NC, NS, NPT, FC), jnp.float32),
    mesh=_MESH,
    compiler_params=pltpu.CompilerParams(needs_layout_passes=False),
    scratch_types=[
        pltpu.VMEM((CROWS, BW), jnp.int32),
        pltpu.VMEM((CROWS, BW), jnp.int32),
        pltpu.VMEM((ECH,), jnp.float32),
        pltpu.VMEM((ECH * 16,), jnp.float32),
        pltpu.VMEM((64, FC), jnp.float32),
        pltpu.VMEM((BW, FC), jnp.float32),
        pltpu.SemaphoreType.DMA,
        pltpu.VMEM_SHARED((NPAD, FC), jnp.float32),
    ],
)
def _sc_scatter(src3d_hbm, dst3d_hbm, hcat_hbm, wall_hbm,
                outp_hbm,
                src_v, dst_v, wt_v, w16_v, g_v, r_v, sem, out_sp):
    c = lax.axis_index("c")
    s = lax.axis_index("s")
    wid = c * NS + s

    def zrow(r, carry):
        def zc(k, carry2):
            r_v[r, pl.ds(k * 16, 16)] = jnp.zeros((16,), jnp.float32)
            return carry2
        lax.fori_loop(0, FC // 16, zc, 0)
        return carry
    lax.fori_loop(0, BW, zrow, 0)

    def zdma(t, carry):
        pltpu.sync_copy(r_v, out_sp.at[pl.ds(s * NPT + t * BW, BW)])
        return carry
    lax.fori_loop(0, NPT // BW, zdma, 0)
    plsc.subcore_barrier()

    iota16 = lax.iota(jnp.int32, 16)

    def chunk(cc, carry):
        pltpu.sync_copy(src3d_hbm.at[wid, pl.ds(cc * CROWS, CROWS)], src_v)
        pltpu.sync_copy(dst3d_hbm.at[wid, pl.ds(cc * CROWS, CROWS)], dst_v)

        def bhead(h, carry2):
            pltpu.sync_copy(
                wall_hbm.at[pl.ds((wid * H + h) * EPT + cc * ECH, ECH)], wt_v)

            def bgrp(i, carry3):
                w = wt_v[pl.ds(i * 16, 16)]
                plsc.store_scatter(w16_v, [(i * 256 + h) + iota16 * 16], w)
                return carry3
            lax.fori_loop(0, ECH // 16, bgrp, 0)
            return carry2
        lax.fori_loop(0, H, bhead, 0)

        def brow(j, carry2):
            def half(bb, carry3):
                pltpu.async_copy(
                    hcat_hbm.at[src_v.at[j, pl.ds(bb * 64, 64)]],
                    g_v, sem).wait()

                def edge(r, carry4):
                    el = j * BW + bb * 64 + r
                    arow = w16_v[pl.ds(el * 16, 16)]
                    for h in range(H):
                        g = g_v[r, pl.ds(h * 16, 16)]
                        r_v[bb * 64 + r, pl.ds(h * 16, 16)] = arow[h] * g
                    return carry4
                lax.fori_loop(0, 64, edge, 0)
                return carry3
            lax.fori_loop(0, 2, half, 0)
            pltpu.sync_copy(r_v, out_sp.at[dst_v.at[j]], add=True)
            return carry2
        lax.fori_loop(0, CROWS, brow, 0)
        return carry
    lax.fori_loop(0, CH, chunk, 0)

    plsc.subcore_barrier()
    pltpu.sync_copy(out_sp.at[pl.ds(s * NPT, NPT)], outp_hbm.at[c, s])


def kernel(x, edge_index, W, a_src, a_dst):
    wcat = jnp.transpose(W, (1, 0, 2)).reshape(IN_F, FC)
    blk = jnp.repeat(jnp.arange(H), OUT_F)
    ams = jnp.where(blk[:, None] == jnp.arange(H)[None, :],
                    a_src.reshape(FC)[:, None], 0.0)
    amd = jnp.where(blk[:, None] == jnp.arange(H)[None, :],
                    a_dst.reshape(FC)[:, None], 0.0)
    sel = jnp.where(jnp.arange(H)[:, None] == (jnp.arange(FC)[None, :] // 16),
                    1.0, 0.0)

    xpad = jnp.pad(x, ((0, NPAD - N), (0, 0)))

    hcat, alphas, alphad = pl.pallas_call(
        _tc_proj_body,
        grid=(GRID1,),
        in_specs=[
            pl.BlockSpec((ROWB, IN_F), lambda i: (i, 0)),
            pl.BlockSpec((IN_F, FC), lambda i: (0, 0)),
            pl.BlockSpec((IN_F, H), lambda i: (0, 0)),
            pl.BlockSpec((IN_F, H), lambda i: (0, 0)),
        ],
        out_specs=[
            pl.BlockSpec((ROWB, FC), lambda i: (i, 0)),
            pl.BlockSpec((H, ROWB), lambda i: (0, i)),
            pl.BlockSpec((H, ROWB), lambda i: (0, i)),
        ],
        out_shape=[
            jax.ShapeDtypeStruct((NPAD, FC), jnp.float32),
            jax.ShapeDtypeStruct((H, NPAD), jnp.float32),
            jax.ShapeDtypeStruct((H, NPAD), jnp.float32),
        ],
    )(xpad, wcat, ams, amd)

    padi = (N + jnp.arange(EPAD - E, dtype=jnp.int32) % (NPAD - N))
    srcf = jnp.concatenate([edge_index[0], padi])
    dstf = jnp.concatenate([edge_index[1], padi])
    src3d = srcf.reshape(NW, NB, BW)
    dst3d = dstf.reshape(NW, NB, BW)

    dpart, wall = _sc_logits(alphas.reshape(H * NPAD),
                             alphad.reshape(H * NPAD), srcf, dstf)
    outp = _sc_scatter(src3d, dst3d, hcat, wall)

    dpart = dpart.reshape(NW, H, NPAD)
    parts = outp.reshape(NC, NPAD, FC)

    out = pl.pallas_call(
        _tc_final_body,
        grid=(GRID1,),
        in_specs=[
            pl.BlockSpec((NW, H, ROWB), lambda i: (0, 0, i)),
            pl.BlockSpec((H, FC), lambda i: (0, 0)),
            pl.BlockSpec((ROWB, FC), lambda i: (i, 0)),
            pl.BlockSpec((ROWB, FC), lambda i: (i, 0)),
            pl.BlockSpec((ROWB, FC), lambda i: (i, 0)),
        ],
        out_specs=pl.BlockSpec((ROWB, FC), lambda i: (i, 0)),
        out_shape=jax.ShapeDtypeStruct((NPAD, FC), jnp.float32),
    )(dpart, sel, parts[0], parts[1], xpad)
    return out[:N]


# speedup vs baseline: 50.3368x; 1.1726x over previous
import functools

import jax
import jax.numpy as jnp
from jax import lax
from jax.experimental import pallas as pl
from jax.experimental.pallas import tpu as pltpu
from jax.experimental.pallas import tpu_sc as plsc

N = 10000
E = 320000
IN_F = 128
OUT_F = 16
H = 8
FC = H * OUT_F

NC = 2
NS = 16
NW = NC * NS

NPAD = 10240
NPT = NPAD // NS

EPAD = NW * NPAD
EPT = EPAD // NW
BW = 128
NB = EPT // BW
CROWS = 8
CH = NB // CROWS
ECH = CROWS * BW

ROWB = 1024
GRID1 = NPAD // ROWB


def _tc_proj_body(x_ref, wcat_ref, ams_ref, amd_ref, h_ref, as_ref, ad_ref):
    xb = x_ref[...]
    hb = jnp.dot(xb, wcat_ref[...], preferred_element_type=jnp.float32)
    h_ref[...] = hb
    dn = (((0,), (1,)), ((), ()))
    as_ref[...] = lax.dot_general(ams_ref[...], hb, dn,
                                  preferred_element_type=jnp.float32)
    ad_ref[...] = lax.dot_general(amd_ref[...], hb, dn,
                                  preferred_element_type=jnp.float32)


def _tc_final_body(d_ref, sel_ref, p0_ref, p1_ref, x_ref, o_ref):
    den8 = jnp.sum(d_ref[...], axis=0)
    den128 = lax.dot_general(den8, sel_ref[...],
                             (((0,), (0,)), ((), ())),
                             preferred_element_type=jnp.float32)
    recip = 1.0 / (den128 + 1e-10)
    o_ref[...] = (p0_ref[...] + p1_ref[...]) * recip + x_ref[...]


_MESH = plsc.VectorSubcoreMesh(core_axis_name="c", subcore_axis_name="s")


@functools.partial(
    pl.kernel,
    out_type=[
        jax.ShapeDtypeStruct((NW * H * NPAD,), jnp.float32),
        jax.ShapeDtypeStruct((NW * H * EPT,), jnp.float32),
    ],
    mesh=_MESH,
    compiler_params=pltpu.CompilerParams(needs_layout_passes=False),
    scratch_types=[
        pltpu.VMEM((EPT,), jnp.int32),
        pltpu.VMEM((EPT,), jnp.int32),
        pltpu.VMEM((NPAD,), jnp.float32),
        pltpu.VMEM((NPAD,), jnp.float32),
        pltpu.VMEM((NPAD,), jnp.float32),
        pltpu.VMEM((EPT,), jnp.float32),
    ],
)
def _sc_logits(alphas_hbm, alphad_hbm, srcf_hbm, dstf_hbm,
               dpart_hbm, wall_hbm,
               srcf_v, dstf_v, as_v, ad_v, den_v, w_v):
    c = lax.axis_index("c")
    s = lax.axis_index("s")
    wid = c * NS + s

    pltpu.sync_copy(srcf_hbm.at[pl.ds(wid * EPT, EPT)], srcf_v)
    pltpu.sync_copy(dstf_hbm.at[pl.ds(wid * EPT, EPT)], dstf_v)

    def head(h, carry):
        pltpu.sync_copy(alphas_hbm.at[pl.ds(h * NPAD, NPAD)], as_v)
        pltpu.sync_copy(alphad_hbm.at[pl.ds(h * NPAD, NPAD)], ad_v)

        def dz(r, carry2):
            den_v[pl.ds(r * 16, 16)] = jnp.zeros((16,), jnp.float32)
            return carry2
        lax.fori_loop(0, NPAD // 16, dz, 0)

        def grp(i, carry2):
            off = i * 16
            s16 = srcf_v[pl.ds(off, 16)]
            d16 = dstf_v[pl.ds(off, 16)]
            a1 = plsc.load_gather(as_v, [s16])
            a2 = plsc.load_gather(ad_v, [d16])
            e = a1 + a2
            e = jnp.where(e > 0.0, e, 0.2 * e)
            w = jnp.exp(e)
            w_v[pl.ds(off, 16)] = w
            plsc.addupdate_scatter(den_v, [d16], w)
            return carry2
        lax.fori_loop(0, EPT // 16, grp, 0)

        pltpu.sync_copy(den_v, dpart_hbm.at[pl.ds((wid * H + h) * NPAD, NPAD)])
        pltpu.sync_copy(w_v, wall_hbm.at[pl.ds((wid * H + h) * EPT, EPT)])
        return carry
    lax.fori_loop(0, H, head, 0)


@functools.partial(
    pl.kernel,
    out_type=jax.ShapeDtypeStruct((NC, NS, NPT, FC), jnp.float32),
    mesh=_MESH,
    compiler_params=pltpu.CompilerParams(needs_layout_passes=False),
    scratch_types=[
        pltpu.VMEM((CROWS, BW), jnp.int32),
        pltpu.VMEM((2 * CROWS, 64), jnp.int32),
        pltpu.VMEM((512,), jnp.float32),
        pltpu.VMEM((512 * 16,), jnp.float32),
        pltpu.VMEM((32, FC), jnp.float32),
        pltpu.VMEM((32, FC), jnp.float32),
        pltpu.VMEM((32, FC), jnp.float32),
        pltpu.VMEM((32, FC), jnp.float32),
        pltpu.VMEM((64, FC), jnp.float32),
        pltpu.VMEM((64, FC), jnp.float32),
        pltpu.SemaphoreType.DMA,
        pltpu.SemaphoreType.DMA,
        pltpu.SemaphoreType.DMA,
        pltpu.SemaphoreType.DMA,
        pltpu.SemaphoreType.DMA,
        pltpu.SemaphoreType.DMA,
        pltpu.VMEM_SHARED((NPAD, FC), jnp.float32),
    ],
)
def _sc_scatter(src3d_hbm, dst3d_hbm, hcat_hbm, wall_hbm,
                outp_hbm,
                src_v, dst_v, wt_v, w16_v, g0_v, g1_v, g2_v, g3_v,
                r0_v, r1_v, sg0, sg1, sg2, sg3, ss0, ss1, out_sp):
    c = lax.axis_index("c")
    s = lax.axis_index("s")
    wid = c * NS + s
    gbufs = ((g0_v, g1_v), (g2_v, g3_v))
    gsems = ((sg0, sg1), (sg2, sg3))
    rbufs = (r0_v, r1_v)
    rsems = (ss0, ss1)

    def zrow(r, carry):
        def zc(k, carry2):
            r0_v[r, pl.ds(k * 16, 16)] = jnp.zeros((16,), jnp.float32)
            return carry2
        lax.fori_loop(0, FC // 16, zc, 0)
        return carry
    lax.fori_loop(0, 64, zrow, 0)

    def zdma(t, carry):
        pltpu.sync_copy(r0_v, out_sp.at[pl.ds(s * NPT + t * 64, 64)])
        return carry
    lax.fori_loop(0, NPT // 64, zdma, 0)
    plsc.subcore_barrier()

    iota16 = lax.iota(jnp.int32, 16)

    def make_edge_fn(rbuf, gbuf, rowoff, base):
        def edge(r, carry):
            arow = w16_v[pl.ds((base + r) * 16, 16)]
            for h in range(H):
                g = gbuf[r, pl.ds(h * 16, 16)]
                rbuf[rowoff + r, pl.ds(h * 16, 16)] = arow[h] * g
            return carry
        return edge

    def gstart(j, off, buf, sem):
        return pltpu.async_copy(
            hcat_hbm.at[src_v.at[j, pl.ds(off, 32)]], buf, sem)

    def chunk(cc, carry):
        pltpu.sync_copy(src3d_hbm.at[wid, pl.ds(cc * CROWS, CROWS)], src_v)
        pltpu.sync_copy(dst3d_hbm.at[wid, pl.ds(cc * 2 * CROWS, 2 * CROWS)],
                        dst_v)

        for half in range(2):
            def bhead(h, carry2):
                pltpu.sync_copy(
                    wall_hbm.at[pl.ds((wid * H + h) * EPT
                                      + cc * ECH + half * 512, 512)], wt_v)

                def bgrp(i, carry3):
                    w = wt_v[pl.ds(i * 16, 16)]
                    plsc.store_scatter(w16_v, [(i * 256 + h) + iota16 * 16], w)
                    return carry3
                lax.fori_loop(0, 512 // 16, bgrp, 0)
                return carry2
            lax.fori_loop(0, H, bhead, 0)

            def uidx(u):
                j = half * 4 + u // 2
                off = (u % 2) * 64
                return j, off

            pend = None
            sc_pend = [None, None]
            for u in range(8):
                p = u % 2
                ga, gb = gbufs[p]
                sa, sb = gsems[p]
                if u == 0:
                    j, off = uidx(0)
                    d_ga = gstart(j, off, ga, sa)
                    d_gb = gstart(j, off + 32, gb, sb)
                else:
                    d_ga, d_gb = pend
                if u < 7:
                    np_ = (u + 1) % 2
                    nga, ngb = gbufs[np_]
                    nsa, nsb = gsems[np_]
                    j, off = uidx(u + 1)
                    pend = (gstart(j, off, nga, nsa),
                            gstart(j, off + 32, ngb, nsb))
                rbuf = rbufs[p]
                if sc_pend[p] is not None:
                    sc_pend[p].wait()
                d_ga.wait()
                lax.fori_loop(0, 32,
                              make_edge_fn(rbuf, ga, 0, u * 64), 0)
                d_gb.wait()
                lax.fori_loop(0, 32,
                              make_edge_fn(rbuf, gb, 32, u * 64 + 32), 0)
                sc_pend[p] = pltpu.async_copy(
                    rbuf, out_sp.at[dst_v.at[half * 8 + u]],
                    rsems[p], add=True)
            sc_pend[0].wait()
            sc_pend[1].wait()
        return carry
    lax.fori_loop(0, CH, chunk, 0)

    plsc.subcore_barrier()
    pltpu.sync_copy(out_sp.at[pl.ds(s * NPT, NPT)], outp_hbm.at[c, s])


def kernel(x, edge_index, W, a_src, a_dst):
    wcat = jnp.transpose(W, (1, 0, 2)).reshape(IN_F, FC)
    blk = jnp.repeat(jnp.arange(H), OUT_F)
    ams = jnp.where(blk[:, None] == jnp.arange(H)[None, :],
                    a_src.reshape(FC)[:, None], 0.0)
    amd = jnp.where(blk[:, None] == jnp.arange(H)[None, :],
                    a_dst.reshape(FC)[:, None], 0.0)
    sel = jnp.where(jnp.arange(H)[:, None] == (jnp.arange(FC)[None, :] // 16),
                    1.0, 0.0)

    xpad = jnp.pad(x, ((0, NPAD - N), (0, 0)))

    hcat, alphas, alphad = pl.pallas_call(
        _tc_proj_body,
        grid=(GRID1,),
        in_specs=[
            pl.BlockSpec((ROWB, IN_F), lambda i: (i, 0)),
            pl.BlockSpec((IN_F, FC), lambda i: (0, 0)),
            pl.BlockSpec((IN_F, H), lambda i: (0, 0)),
            pl.BlockSpec((IN_F, H), lambda i: (0, 0)),
        ],
        out_specs=[
            pl.BlockSpec((ROWB, FC), lambda i: (i, 0)),
            pl.BlockSpec((H, ROWB), lambda i: (0, i)),
            pl.BlockSpec((H, ROWB), lambda i: (0, i)),
        ],
        out_shape=[
            jax.ShapeDtypeStruct((NPAD, FC), jnp.float32),
            jax.ShapeDtypeStruct((H, NPAD), jnp.float32),
            jax.ShapeDtypeStruct((H, NPAD), jnp.float32),
        ],
    )(xpad, wcat, ams, amd)

    padi = (N + jnp.arange(EPAD - E, dtype=jnp.int32) % (NPAD - N))
    srcf = jnp.concatenate([edge_index[0], padi])
    dstf = jnp.concatenate([edge_index[1], padi])
    src3d = srcf.reshape(NW, NB, BW)
    dst3d = dstf.reshape(NW, 2 * NB, 64)

    dpart, wall = _sc_logits(alphas.reshape(H * NPAD),
                             alphad.reshape(H * NPAD), srcf, dstf)
    outp = _sc_scatter(src3d, dst3d, hcat, wall)

    dpart = dpart.reshape(NW, H, NPAD)
    parts = outp.reshape(NC, NPAD, FC)

    out = pl.pallas_call(
        _tc_final_body,
        grid=(GRID1,),
        in_specs=[
            pl.BlockSpec((NW, H, ROWB), lambda i: (0, 0, i)),
            pl.BlockSpec((H, FC), lambda i: (0, 0)),
            pl.BlockSpec((ROWB, FC), lambda i: (i, 0)),
            pl.BlockSpec((ROWB, FC), lambda i: (i, 0)),
            pl.BlockSpec((ROWB, FC), lambda i: (i, 0)),
        ],
        out_specs=pl.BlockSpec((ROWB, FC), lambda i: (i, 0)),
        out_shape=jax.ShapeDtypeStruct((NPAD, FC), jnp.float32),
    )(dpart, sel, parts[0], parts[1], xpad)
    return out[:N]


# speedup vs baseline: 50.6489x; 1.0062x over previous
import functools

import jax
import jax.numpy as jnp
from jax import lax
from jax.experimental import pallas as pl
from jax.experimental.pallas import tpu as pltpu
from jax.experimental.pallas import tpu_sc as plsc

N = 10000
E = 320000
IN_F = 128
OUT_F = 16
H = 8
FC = H * OUT_F

NC = 2
NS = 16
NW = NC * NS

NPAD = 10240
NPT = NPAD // NS

EPAD = NW * NPAD
EPT = EPAD // NW
BW = 128
NB = EPT // BW
CROWS = 8
CH = NB // CROWS
ECH = CROWS * BW

ROWB = 1024
GRID1 = NPAD // ROWB


def _tc_proj_body(x_ref, wcat_ref, ams_ref, amd_ref, h_ref, as_ref, ad_ref):
    xb = x_ref[...]
    hb = jnp.dot(xb, wcat_ref[...], preferred_element_type=jnp.float32)
    h_ref[...] = hb
    dn = (((0,), (1,)), ((), ()))
    as_ref[...] = lax.dot_general(ams_ref[...], hb, dn,
                                  preferred_element_type=jnp.float32)
    ad_ref[...] = lax.dot_general(amd_ref[...], hb, dn,
                                  preferred_element_type=jnp.float32)


def _tc_final_body(d_ref, sel_ref, p0_ref, p1_ref, x_ref, o_ref):
    den8 = jnp.sum(d_ref[...], axis=0)
    den128 = lax.dot_general(den8, sel_ref[...],
                             (((0,), (0,)), ((), ())),
                             preferred_element_type=jnp.float32)
    recip = 1.0 / (den128 + 1e-10)
    o_ref[...] = (p0_ref[...] + p1_ref[...]) * recip + x_ref[...]


_MESH = plsc.VectorSubcoreMesh(core_axis_name="c", subcore_axis_name="s")


@functools.partial(
    pl.kernel,
    out_type=[
        jax.ShapeDtypeStruct((NW * H * NPAD,), jnp.float32),
        jax.ShapeDtypeStruct((NW * H * EPT,), jnp.float32),
    ],
    mesh=_MESH,
    compiler_params=pltpu.CompilerParams(needs_layout_passes=False),
    scratch_types=[
        pltpu.VMEM((EPT,), jnp.int32),
        pltpu.VMEM((EPT,), jnp.int32),
        pltpu.VMEM((NPAD,), jnp.float32),
        pltpu.VMEM((NPAD,), jnp.float32),
        pltpu.VMEM((NPAD,), jnp.float32),
        pltpu.VMEM((EPT,), jnp.float32),
    ],
)
def _sc_logits(alphas_hbm, alphad_hbm, srcf_hbm, dstf_hbm,
               dpart_hbm, wall_hbm,
               srcf_v, dstf_v, as_v, ad_v, den_v, w_v):
    c = lax.axis_index("c")
    s = lax.axis_index("s")
    wid = c * NS + s

    pltpu.sync_copy(srcf_hbm.at[pl.ds(wid * EPT, EPT)], srcf_v)
    pltpu.sync_copy(dstf_hbm.at[pl.ds(wid * EPT, EPT)], dstf_v)

    def head(h, carry):
        pltpu.sync_copy(alphas_hbm.at[pl.ds(h * NPAD, NPAD)], as_v)
        pltpu.sync_copy(alphad_hbm.at[pl.ds(h * NPAD, NPAD)], ad_v)

        def dz(r, carry2):
            den_v[pl.ds(r * 16, 16)] = jnp.zeros((16,), jnp.float32)
            return carry2
        lax.fori_loop(0, NPAD // 16, dz, 0)

        def grp(i, carry2):
            off = i * 16
            s16 = srcf_v[pl.ds(off, 16)]
            d16 = dstf_v[pl.ds(off, 16)]
            a1 = plsc.load_gather(as_v, [s16])
            a2 = plsc.load_gather(ad_v, [d16])
            e = a1 + a2
            e = jnp.where(e > 0.0, e, 0.2 * e)
            w = jnp.exp(e)
            w_v[pl.ds(off, 16)] = w
            plsc.addupdate_scatter(den_v, [d16], w)
            return carry2
        lax.fori_loop(0, EPT // 16, grp, 0)

        pltpu.sync_copy(den_v, dpart_hbm.at[pl.ds((wid * H + h) * NPAD, NPAD)])
        pltpu.sync_copy(w_v, wall_hbm.at[pl.ds((wid * H + h) * EPT, EPT)])
        return carry
    lax.fori_loop(0, H, head, 0)


_BCAST_DN = lax.GatherDimensionNumbers(
    offset_dims=(), collapsed_slice_dims=(0,), start_index_map=(0,))


@functools.partial(
    pl.kernel,
    out_type=jax.ShapeDtypeStruct((NC, NS, NPT, FC), jnp.float32),
    mesh=_MESH,
    compiler_params=pltpu.CompilerParams(needs_layout_passes=False),
    scratch_types=[
        pltpu.VMEM((CROWS, BW), jnp.int32),
        pltpu.VMEM((2 * CROWS, 64), jnp.int32),
        pltpu.VMEM((512,), jnp.float32),
        pltpu.VMEM((512 * 16,), jnp.float32),
        pltpu.VMEM((64, FC), jnp.float32),
        pltpu.VMEM((64, FC), jnp.float32),
        pltpu.VMEM((64, FC), jnp.float32),
        pltpu.VMEM((64, FC), jnp.float32),
        pltpu.SemaphoreType.DMA,
        pltpu.SemaphoreType.DMA,
        pltpu.SemaphoreType.DMA,
        pltpu.SemaphoreType.DMA,
        pltpu.VMEM_SHARED((NPAD, FC), jnp.float32),
    ],
)
def _sc_scatter(src3d_hbm, dst3d_hbm, hcat_hbm, wall_hbm,
                outp_hbm,
                src_v, dst_v, wt_v, w16_v, g0_v, g1_v,
                r0_v, r1_v, sg0, sg1, ss0, ss1, out_sp):
    c = lax.axis_index("c")
    s = lax.axis_index("s")
    wid = c * NS + s
    gbufs = (g0_v, g1_v)
    gsems = (sg0, sg1)
    rbufs = (r0_v, r1_v)
    rsems = (ss0, ss1)

    def zrow(r, carry):
        def zc(k, carry2):
            r0_v[r, pl.ds(k * 16, 16)] = jnp.zeros((16,), jnp.float32)
            return carry2
        lax.fori_loop(0, FC // 16, zc, 0)
        return carry
    lax.fori_loop(0, 64, zrow, 0)

    def zdma(t, carry):
        pltpu.sync_copy(r0_v, out_sp.at[pl.ds(s * NPT + t * 64, 64)])
        return carry
    lax.fori_loop(0, NPT // 64, zdma, 0)
    plsc.subcore_barrier()

    iota16 = lax.iota(jnp.int32, 16)
    hidx = [jnp.full((16, 1), h, jnp.int32) for h in range(H)]

    def make_edge_fn(rbuf, gbuf, base):
        def edge(r, carry):
            arow = w16_v[pl.ds((base + r) * 16, 16)]
            for h in range(H):
                m = lax.gather(arow, hidx[h], _BCAST_DN, (1,),
                               mode=lax.GatherScatterMode.PROMISE_IN_BOUNDS)
                g = gbuf[r, pl.ds(h * 16, 16)]
                rbuf[r, pl.ds(h * 16, 16)] = m * g
            return carry
        return edge

    def gstart(u, half, buf, sem):
        j = half * 4 + u // 2
        off = (u % 2) * 64
        return pltpu.async_copy(
            hcat_hbm.at[src_v.at[j, pl.ds(off, 64)]], buf, sem)

    def chunk(cc, carry):
        pltpu.sync_copy(src3d_hbm.at[wid, pl.ds(cc * CROWS, CROWS)], src_v)
        pltpu.sync_copy(dst3d_hbm.at[wid, pl.ds(cc * 2 * CROWS, 2 * CROWS)],
                        dst_v)

        for half in range(2):
            def bhead(h, carry2):
                pltpu.sync_copy(
                    wall_hbm.at[pl.ds((wid * H + h) * EPT
                                      + cc * ECH + half * 512, 512)], wt_v)

                def bgrp(i, carry3):
                    w = wt_v[pl.ds(i * 16, 16)]
                    plsc.store_scatter(w16_v, [(i * 256 + h) + iota16 * 16], w)
                    return carry3
                lax.fori_loop(0, 512 // 16, bgrp, 0)
                return carry2
            lax.fori_loop(0, H, bhead, 0)

            pend = None
            sc_pend = [None, None]
            for u in range(8):
                pa = u % 2
                if u == 0:
                    d_g = gstart(0, half, gbufs[0], gsems[0])
                else:
                    d_g = pend
                if u < 7:
                    pend = gstart(u + 1, half, gbufs[(u + 1) % 2],
                                  gsems[(u + 1) % 2])
                rbuf = rbufs[pa]
                if sc_pend[pa] is not None:
                    sc_pend[pa].wait()
                d_g.wait()
                lax.fori_loop(0, 64, make_edge_fn(rbuf, gbufs[pa], u * 64), 0)
                sc_pend[pa] = pltpu.async_copy(
                    rbuf, out_sp.at[dst_v.at[half * 8 + u]],
                    rsems[pa], add=True)
            sc_pend[0].wait()
            sc_pend[1].wait()
        return carry
    lax.fori_loop(0, CH, chunk, 0)

    plsc.subcore_barrier()
    pltpu.sync_copy(out_sp.at[pl.ds(s * NPT, NPT)], outp_hbm.at[c, s])


def kernel(x, edge_index, W, a_src, a_dst):
    wcat = jnp.transpose(W, (1, 0, 2)).reshape(IN_F, FC)
    blk = jnp.repeat(jnp.arange(H), OUT_F)
    ams = jnp.where(blk[:, None] == jnp.arange(H)[None, :],
                    a_src.reshape(FC)[:, None], 0.0)
    amd = jnp.where(blk[:, None] == jnp.arange(H)[None, :],
                    a_dst.reshape(FC)[:, None], 0.0)
    sel = jnp.where(jnp.arange(H)[:, None] == (jnp.arange(FC)[None, :] // 16),
                    1.0, 0.0)

    xpad = jnp.pad(x, ((0, NPAD - N), (0, 0)))

    hcat, alphas, alphad = pl.pallas_call(
        _tc_proj_body,
        grid=(GRID1,),
        in_specs=[
            pl.BlockSpec((ROWB, IN_F), lambda i: (i, 0)),
            pl.BlockSpec((IN_F, FC), lambda i: (0, 0)),
            pl.BlockSpec((IN_F, H), lambda i: (0, 0)),
            pl.BlockSpec((IN_F, H), lambda i: (0, 0)),
        ],
        out_specs=[
            pl.BlockSpec((ROWB, FC), lambda i: (i, 0)),
            pl.BlockSpec((H, ROWB), lambda i: (0, i)),
            pl.BlockSpec((H, ROWB), lambda i: (0, i)),
        ],
        out_shape=[
            jax.ShapeDtypeStruct((NPAD, FC), jnp.float32),
            jax.ShapeDtypeStruct((H, NPAD), jnp.float32),
            jax.ShapeDtypeStruct((H, NPAD), jnp.float32),
        ],
    )(xpad, wcat, ams, amd)

    padi = (N + jnp.arange(EPAD - E, dtype=jnp.int32) % (NPAD - N))
    srcf = jnp.concatenate([edge_index[0], padi])
    dstf = jnp.concatenate([edge_index[1], padi])
    src3d = srcf.reshape(NW, NB, BW)
    dst3d = dstf.reshape(NW, 2 * NB, 64)

    dpart, wall = _sc_logits(alphas.reshape(H * NPAD),
                             alphad.reshape(H * NPAD), srcf, dstf)
    outp = _sc_scatter(src3d, dst3d, hcat, wall)

    dpart = dpart.reshape(NW, H, NPAD)
    parts = outp.reshape(NC, NPAD, FC)

    out = pl.pallas_call(
        _tc_final_body,
        grid=(GRID1,),
        in_specs=[
            pl.BlockSpec((NW, H, ROWB), lambda i: (0, 0, i)),
            pl.BlockSpec((H, FC), lambda i: (0, 0)),
            pl.BlockSpec((ROWB, FC), lambda i: (i, 0)),
            pl.BlockSpec((ROWB, FC), lambda i: (i, 0)),
            pl.BlockSpec((ROWB, FC), lambda i: (i, 0)),
        ],
        out_specs=pl.BlockSpec((ROWB, FC), lambda i: (i, 0)),
        out_shape=jax.ShapeDtypeStruct((NPAD, FC), jnp.float32),
    )(dpart, sel, parts[0], parts[1], xpad)
    return out[:N]


# speedup vs baseline: 55.6143x; 1.0980x over previous
import functools

import jax
import jax.numpy as jnp
from jax import lax
from jax.experimental import pallas as pl
from jax.experimental.pallas import tpu as pltpu
from jax.experimental.pallas import tpu_sc as plsc

N = 10000
E = 320000
IN_F = 128
OUT_F = 16
H = 8
FC = H * OUT_F

NC = 2
NS = 16
NW = NC * NS

NPAD = 10240
NPT = NPAD // NS

EPAD = NW * NPAD
EPT = EPAD // NW
BW = 128
NB = EPT // BW
CROWS = 8
CH = NB // CROWS
ECH = CROWS * BW

ROWB = 1024
GRID1 = NPAD // ROWB


def _tc_proj_body(x_ref, wcat_ref, ams_ref, amd_ref, h_ref, as_ref, ad_ref):
    xb = x_ref[...]
    hb = jnp.dot(xb, wcat_ref[...], preferred_element_type=jnp.float32)
    h_ref[...] = hb
    dn = (((0,), (1,)), ((), ()))
    as_ref[...] = lax.dot_general(ams_ref[...], hb, dn,
                                  preferred_element_type=jnp.float32)
    ad_ref[...] = lax.dot_general(amd_ref[...], hb, dn,
                                  preferred_element_type=jnp.float32)


def _tc_final_body(d_ref, sel_ref, p0_ref, p1_ref, x_ref, o_ref):
    den8 = jnp.sum(d_ref[...], axis=0)
    den128 = lax.dot_general(den8, sel_ref[...],
                             (((0,), (0,)), ((), ())),
                             preferred_element_type=jnp.float32)
    recip = 1.0 / (den128 + 1e-10)
    o_ref[...] = (p0_ref[...] + p1_ref[...]) * recip + x_ref[...]


_MESH = plsc.VectorSubcoreMesh(core_axis_name="c", subcore_axis_name="s")


@functools.partial(
    pl.kernel,
    out_type=[
        jax.ShapeDtypeStruct((NW * H * NPAD,), jnp.float32),
        jax.ShapeDtypeStruct((NW * H * EPT,), jnp.float32),
    ],
    mesh=_MESH,
    compiler_params=pltpu.CompilerParams(needs_layout_passes=False),
    scratch_types=[
        pltpu.VMEM((EPT,), jnp.int32),
        pltpu.VMEM((EPT,), jnp.int32),
        pltpu.VMEM((NPAD,), jnp.float32),
        pltpu.VMEM((NPAD,), jnp.float32),
        pltpu.VMEM((NPAD,), jnp.float32),
        pltpu.VMEM((EPT,), jnp.float32),
    ],
)
def _sc_logits(alphas_hbm, alphad_hbm, srcf_hbm, dstf_hbm,
               dpart_hbm, wall_hbm,
               srcf_v, dstf_v, as_v, ad_v, den_v, w_v):
    c = lax.axis_index("c")
    s = lax.axis_index("s")
    wid = c * NS + s

    pltpu.sync_copy(srcf_hbm.at[pl.ds(wid * EPT, EPT)], srcf_v)
    pltpu.sync_copy(dstf_hbm.at[pl.ds(wid * EPT, EPT)], dstf_v)

    def head(h, carry):
        pltpu.sync_copy(alphas_hbm.at[pl.ds(h * NPAD, NPAD)], as_v)
        pltpu.sync_copy(alphad_hbm.at[pl.ds(h * NPAD, NPAD)], ad_v)

        def dz(r, carry2):
            den_v[pl.ds(r * 16, 16)] = jnp.zeros((16,), jnp.float32)
            return carry2
        lax.fori_loop(0, NPAD // 16, dz, 0)

        def grp(i, carry2):
            off = i * 16
            s16 = srcf_v[pl.ds(off, 16)]
            d16 = dstf_v[pl.ds(off, 16)]
            a1 = plsc.load_gather(as_v, [s16])
            a2 = plsc.load_gather(ad_v, [d16])
            e = a1 + a2
            e = jnp.where(e > 0.0, e, 0.2 * e)
            w = jnp.exp(e)
            w_v[pl.ds(off, 16)] = w
            plsc.addupdate_scatter(den_v, [d16], w)
            return carry2
        lax.fori_loop(0, EPT // 16, grp, 0)

        pltpu.sync_copy(den_v, dpart_hbm.at[pl.ds((wid * H + h) * NPAD, NPAD)])
        pltpu.sync_copy(w_v, wall_hbm.at[pl.ds((wid * H + h) * EPT, EPT)])
        return carry
    lax.fori_loop(0, H, head, 0)


_BCAST_DN = lax.GatherDimensionNumbers(
    offset_dims=(), collapsed_slice_dims=(0,), start_index_map=(0,))


@functools.partial(
    pl.kernel,
    out_type=jax.ShapeDtypeStruct((NC, NS, NPT, FC), jnp.float32),
    mesh=_MESH,
    compiler_params=pltpu.CompilerParams(needs_layout_passes=False),
    scratch_types=[
        pltpu.VMEM((CROWS, BW), jnp.int32),
        pltpu.VMEM((2 * CROWS, 64), jnp.int32),
        pltpu.VMEM((H, 512), jnp.float32),
        pltpu.VMEM((512 * 16,), jnp.float32),
        pltpu.VMEM((64, FC), jnp.float32),
        pltpu.VMEM((64, FC), jnp.float32),
        pltpu.VMEM((64, FC), jnp.float32),
        pltpu.VMEM((64, FC), jnp.float32),
        pltpu.SemaphoreType.DMA,
        pltpu.SemaphoreType.DMA,
        pltpu.SemaphoreType.DMA,
        pltpu.SemaphoreType.DMA,
        pltpu.VMEM_SHARED((NPAD, FC), jnp.float32),
    ],
)
def _sc_scatter(src3d_hbm, dst3d_hbm, hcat_hbm, wall_hbm,
                outp_hbm,
                src_v, dst_v, wt_v, w16_v, g0_v, g1_v,
                r0_v, r1_v, sg0, sg1, ss0, ss1, out_sp):
    c = lax.axis_index("c")
    s = lax.axis_index("s")
    wid = c * NS + s
    gbufs = (g0_v, g1_v)
    gsems = (sg0, sg1)
    rbufs = (r0_v, r1_v)
    rsems = (ss0, ss1)

    def zrow(r, carry):
        def zc(k, carry2):
            r0_v[r, pl.ds(k * 16, 16)] = jnp.zeros((16,), jnp.float32)
            return carry2
        lax.fori_loop(0, FC // 16, zc, 0)
        return carry
    lax.fori_loop(0, 64, zrow, 0)

    def zdma(t, carry):
        pltpu.sync_copy(r0_v, out_sp.at[pl.ds(s * NPT + t * 64, 64)])
        return carry
    lax.fori_loop(0, NPT // 64, zdma, 0)
    plsc.subcore_barrier()

    iota16 = lax.iota(jnp.int32, 16)
    hidx = [jnp.full((16, 1), h, jnp.int32) for h in range(H)]

    def make_edge_fn(rbuf, gbuf, base):
        def edge(r, carry):
            arow = w16_v[pl.ds((base + r) * 16, 16)]
            for h in range(H):
                m = lax.gather(arow, hidx[h], _BCAST_DN, (1,),
                               mode=lax.GatherScatterMode.PROMISE_IN_BOUNDS)
                g = gbuf[r, pl.ds(h * 16, 16)]
                rbuf[r, pl.ds(h * 16, 16)] = m * g
            return carry
        return edge

    def gstart(u, half, buf, sem):
        j = half * 4 + u // 2
        off = (u % 2) * 64
        return pltpu.async_copy(
            hcat_hbm.at[src_v.at[j, pl.ds(off, 64)]], buf, sem)

    def chunk(cc, carry):
        pltpu.sync_copy(src3d_hbm.at[wid, pl.ds(cc * CROWS, CROWS)], src_v)
        pltpu.sync_copy(dst3d_hbm.at[wid, pl.ds(cc * 2 * CROWS, 2 * CROWS)],
                        dst_v)

        for half in range(2):
            wdescs = [
                pltpu.async_copy(
                    wall_hbm.at[pl.ds((wid * H + h) * EPT
                                      + cc * ECH + half * 512, 512)],
                    wt_v.at[h], sg0)
                for h in range(H)]
            for d in wdescs:
                d.wait()

            def bhead(h, carry2):
                def bgrp(i, carry3):
                    w = wt_v[h, pl.ds(i * 16, 16)]
                    plsc.store_scatter(w16_v, [(i * 256 + h) + iota16 * 16], w)
                    return carry3
                lax.fori_loop(0, 512 // 16, bgrp, 0)
                return carry2
            lax.fori_loop(0, H, bhead, 0)

            pend = None
            sc_pend = [None, None]
            for u in range(8):
                pa = u % 2
                if u == 0:
                    d_g = gstart(0, half, gbufs[0], gsems[0])
                else:
                    d_g = pend
                if u < 7:
                    pend = gstart(u + 1, half, gbufs[(u + 1) % 2],
                                  gsems[(u + 1) % 2])
                rbuf = rbufs[pa]
                if sc_pend[pa] is not None:
                    sc_pend[pa].wait()
                d_g.wait()
                lax.fori_loop(0, 64, make_edge_fn(rbuf, gbufs[pa], u * 64), 0)
                sc_pend[pa] = pltpu.async_copy(
                    rbuf, out_sp.at[dst_v.at[half * 8 + u]],
                    rsems[pa], add=True)
            sc_pend[0].wait()
            sc_pend[1].wait()
        return carry
    lax.fori_loop(0, CH, chunk, 0)

    plsc.subcore_barrier()
    pltpu.sync_copy(out_sp.at[pl.ds(s * NPT, NPT)], outp_hbm.at[c, s])


def kernel(x, edge_index, W, a_src, a_dst):
    wcat = jnp.transpose(W, (1, 0, 2)).reshape(IN_F, FC)
    blk = jnp.repeat(jnp.arange(H), OUT_F)
    ams = jnp.where(blk[:, None] == jnp.arange(H)[None, :],
                    a_src.reshape(FC)[:, None], 0.0)
    amd = jnp.where(blk[:, None] == jnp.arange(H)[None, :],
                    a_dst.reshape(FC)[:, None], 0.0)
    sel = jnp.where(jnp.arange(H)[:, None] == (jnp.arange(FC)[None, :] // 16),
                    1.0, 0.0)

    xpad = jnp.pad(x, ((0, NPAD - N), (0, 0)))

    hcat, alphas, alphad = pl.pallas_call(
        _tc_proj_body,
        grid=(GRID1,),
        in_specs=[
            pl.BlockSpec((ROWB, IN_F), lambda i: (i, 0)),
            pl.BlockSpec((IN_F, FC), lambda i: (0, 0)),
            pl.BlockSpec((IN_F, H), lambda i: (0, 0)),
            pl.BlockSpec((IN_F, H), lambda i: (0, 0)),
        ],
        out_specs=[
            pl.BlockSpec((ROWB, FC), lambda i: (i, 0)),
            pl.BlockSpec((H, ROWB), lambda i: (0, i)),
            pl.BlockSpec((H, ROWB), lambda i: (0, i)),
        ],
        out_shape=[
            jax.ShapeDtypeStruct((NPAD, FC), jnp.float32),
            jax.ShapeDtypeStruct((H, NPAD), jnp.float32),
            jax.ShapeDtypeStruct((H, NPAD), jnp.float32),
        ],
    )(xpad, wcat, ams, amd)

    padi = (N + jnp.arange(EPAD - E, dtype=jnp.int32) % (NPAD - N))
    srcf = jnp.concatenate([edge_index[0], padi])
    dstf = jnp.concatenate([edge_index[1], padi])
    src3d = srcf.reshape(NW, NB, BW)
    dst3d = dstf.reshape(NW, 2 * NB, 64)

    dpart, wall = _sc_logits(alphas.reshape(H * NPAD),
                             alphad.reshape(H * NPAD), srcf, dstf)
    outp = _sc_scatter(src3d, dst3d, hcat, wall)

    dpart = dpart.reshape(NW, H, NPAD)
    parts = outp.reshape(NC, NPAD, FC)

    out = pl.pallas_call(
        _tc_final_body,
        grid=(GRID1,),
        in_specs=[
            pl.BlockSpec((NW, H, ROWB), lambda i: (0, 0, i)),
            pl.BlockSpec((H, FC), lambda i: (0, 0)),
            pl.BlockSpec((ROWB, FC), lambda i: (i, 0)),
            pl.BlockSpec((ROWB, FC), lambda i: (i, 0)),
            pl.BlockSpec((ROWB, FC), lambda i: (i, 0)),
        ],
        out_specs=pl.BlockSpec((ROWB, FC), lambda i: (i, 0)),
        out_shape=jax.ShapeDtypeStruct((NPAD, FC), jnp.float32),
    )(dpart, sel, parts[0], parts[1], xpad)
    return out[:N]


# speedup vs baseline: 58.0388x; 1.0436x over previous
import functools

import jax
import jax.numpy as jnp
from jax import lax
from jax.experimental import pallas as pl
from jax.experimental.pallas import tpu as pltpu
from jax.experimental.pallas import tpu_sc as plsc

N = 10000
E = 320000
IN_F = 128
OUT_F = 16
H = 8
FC = H * OUT_F

NC = 2
NS = 16
NW = NC * NS

NPAD = 10240
NPT = NPAD // NS

EPAD = NW * NPAD
EPT = EPAD // NW
BW = 128
NB = EPT // BW
CROWS = 8
CH = NB // CROWS
ECH = CROWS * BW

ROWB = 1024
GRID1 = NPAD // ROWB


def _tc_proj_body(x_ref, wcat_ref, ams_ref, amd_ref, h_ref, as_ref, ad_ref):
    xb = x_ref[...]
    hb = jnp.dot(xb, wcat_ref[...], preferred_element_type=jnp.float32)
    h_ref[...] = hb
    dn = (((0,), (1,)), ((), ()))
    as_ref[...] = lax.dot_general(ams_ref[...], hb, dn,
                                  preferred_element_type=jnp.float32)
    ad_ref[...] = lax.dot_general(amd_ref[...], hb, dn,
                                  preferred_element_type=jnp.float32)


def _tc_final_body(d_ref, sel_ref, p0_ref, p1_ref, x_ref, o_ref):
    den8 = jnp.sum(d_ref[...], axis=0)
    den128 = lax.dot_general(den8, sel_ref[...],
                             (((0,), (0,)), ((), ())),
                             preferred_element_type=jnp.float32)
    recip = 1.0 / (den128 + 1e-10)
    o_ref[...] = (p0_ref[...] + p1_ref[...]) * recip + x_ref[...]


_MESH = plsc.VectorSubcoreMesh(core_axis_name="c", subcore_axis_name="s")


@functools.partial(
    pl.kernel,
    out_type=[
        jax.ShapeDtypeStruct((NW * H * NPAD,), jnp.float32),
        jax.ShapeDtypeStruct((NW * H * EPT,), jnp.float32),
    ],
    mesh=_MESH,
    compiler_params=pltpu.CompilerParams(needs_layout_passes=False),
    scratch_types=[
        pltpu.VMEM((EPT,), jnp.int32),
        pltpu.VMEM((EPT,), jnp.int32),
        pltpu.VMEM((NPAD,), jnp.float32),
        pltpu.VMEM((NPAD,), jnp.float32),
        pltpu.VMEM((NPAD,), jnp.float32),
        pltpu.VMEM((NPAD,), jnp.float32),
        pltpu.VMEM((NPAD,), jnp.float32),
        pltpu.VMEM((NPAD,), jnp.float32),
        pltpu.VMEM((EPT,), jnp.float32),
        pltpu.VMEM((EPT,), jnp.float32),
        pltpu.SemaphoreType.DMA,
        pltpu.SemaphoreType.DMA,
        pltpu.SemaphoreType.DMA,
        pltpu.SemaphoreType.DMA,
    ],
)
def _sc_logits(alphas_hbm, alphad_hbm, srcf_hbm, dstf_hbm,
               dpart_hbm, wall_hbm,
               srcf_v, dstf_v, as0_v, as1_v, ad0_v, ad1_v,
               den0_v, den1_v, w0_v, w1_v, sa0, sa1, sd0, sd1):
    c = lax.axis_index("c")
    s = lax.axis_index("s")
    wid = c * NS + s
    asb = (as0_v, as1_v)
    adb = (ad0_v, ad1_v)
    denb = (den0_v, den1_v)
    wb = (w0_v, w1_v)
    sab = (sa0, sa1)
    sdb = (sd0, sd1)

    pltpu.sync_copy(srcf_hbm.at[pl.ds(wid * EPT, EPT)], srcf_v)
    pltpu.sync_copy(dstf_hbm.at[pl.ds(wid * EPT, EPT)], dstf_v)

    def aload(h, p):
        return (pltpu.async_copy(alphas_hbm.at[pl.ds(h * NPAD, NPAD)],
                                 asb[p], sab[p]),
                pltpu.async_copy(alphad_hbm.at[pl.ds(h * NPAD, NPAD)],
                                 adb[p], sab[p]))

    pend = aload(0, 0)
    dump_pend = [None, None]
    for h in range(H):
        p = h % 2
        d1, d2 = pend
        if h < H - 1:
            pend = aload(h + 1, 1 - p)
        d1.wait()
        d2.wait()
        if dump_pend[p] is not None:
            dump_pend[p][0].wait()
            dump_pend[p][1].wait()
        den_v, w_v, as_v, ad_v = denb[p], wb[p], asb[p], adb[p]

        def dz(r, carry2):
            den_v[pl.ds(r * 16, 16)] = jnp.zeros((16,), jnp.float32)
            return carry2
        lax.fori_loop(0, NPAD // 16, dz, 0)

        def grp(i, carry2):
            off = i * 16
            s16 = srcf_v[pl.ds(off, 16)]
            d16 = dstf_v[pl.ds(off, 16)]
            a1 = plsc.load_gather(as_v, [s16])
            a2 = plsc.load_gather(ad_v, [d16])
            e = a1 + a2
            e = jnp.where(e > 0.0, e, 0.2 * e)
            w = jnp.exp(e)
            w_v[pl.ds(off, 16)] = w
            plsc.addupdate_scatter(den_v, [d16], w)
            return carry2
        lax.fori_loop(0, EPT // 16, grp, 0)

        dump_pend[p] = (
            pltpu.async_copy(
                den_v, dpart_hbm.at[pl.ds((wid * H + h) * NPAD, NPAD)],
                sdb[p]),
            pltpu.async_copy(
                w_v, wall_hbm.at[pl.ds((wid * H + h) * EPT, EPT)], sdb[p]))
    for dp in dump_pend:
        dp[0].wait()
        dp[1].wait()


_BCAST_DN = lax.GatherDimensionNumbers(
    offset_dims=(), collapsed_slice_dims=(0,), start_index_map=(0,))


@functools.partial(
    pl.kernel,
    out_type=jax.ShapeDtypeStruct((NC, NS, NPT, FC), jnp.float32),
    mesh=_MESH,
    compiler_params=pltpu.CompilerParams(needs_layout_passes=False),
    scratch_types=[
        pltpu.VMEM((CROWS, BW), jnp.int32),
        pltpu.VMEM((2 * CROWS, 64), jnp.int32),
        pltpu.VMEM((H, 512), jnp.float32),
        pltpu.VMEM((512 * 16,), jnp.float32),
        pltpu.VMEM((64, FC), jnp.float32),
        pltpu.VMEM((64, FC), jnp.float32),
        pltpu.VMEM((64, FC), jnp.float32),
        pltpu.VMEM((64, FC), jnp.float32),
        pltpu.SemaphoreType.DMA,
        pltpu.SemaphoreType.DMA,
        pltpu.SemaphoreType.DMA,
        pltpu.SemaphoreType.DMA,
        pltpu.VMEM_SHARED((NPAD, FC), jnp.float32),
    ],
)
def _sc_scatter(src3d_hbm, dst3d_hbm, hcat_hbm, wall_hbm,
                outp_hbm,
                src_v, dst_v, wt_v, w16_v, g0_v, g1_v,
                r0_v, r1_v, sg0, sg1, ss0, ss1, out_sp):
    c = lax.axis_index("c")
    s = lax.axis_index("s")
    wid = c * NS + s
    gbufs = (g0_v, g1_v)
    gsems = (sg0, sg1)
    rbufs = (r0_v, r1_v)
    rsems = (ss0, ss1)

    def zrow(r, carry):
        def zc(k, carry2):
            r0_v[r, pl.ds(k * 16, 16)] = jnp.zeros((16,), jnp.float32)
            return carry2
        lax.fori_loop(0, FC // 16, zc, 0)
        return carry
    lax.fori_loop(0, 64, zrow, 0)

    def zdma(t, carry):
        pltpu.sync_copy(r0_v, out_sp.at[pl.ds(s * NPT + t * 64, 64)])
        return carry
    lax.fori_loop(0, NPT // 64, zdma, 0)
    plsc.subcore_barrier()

    iota16 = lax.iota(jnp.int32, 16)
    hidx = [jnp.full((16, 1), h, jnp.int32) for h in range(H)]

    def make_edge_fn(rbuf, gbuf, base):
        def edge(r, carry):
            arow = w16_v[pl.ds((base + r) * 16, 16)]
            for h in range(H):
                m = lax.gather(arow, hidx[h], _BCAST_DN, (1,),
                               mode=lax.GatherScatterMode.PROMISE_IN_BOUNDS)
                g = gbuf[r, pl.ds(h * 16, 16)]
                rbuf[r, pl.ds(h * 16, 16)] = m * g
            return carry
        return edge

    def gstart(u, half, buf, sem):
        j = half * 4 + u // 2
        off = (u % 2) * 64
        return pltpu.async_copy(
            hcat_hbm.at[src_v.at[j, pl.ds(off, 64)]], buf, sem)

    def chunk(cc, carry):
        pltpu.sync_copy(src3d_hbm.at[wid, pl.ds(cc * CROWS, CROWS)], src_v)
        pltpu.sync_copy(dst3d_hbm.at[wid, pl.ds(cc * 2 * CROWS, 2 * CROWS)],
                        dst_v)

        for half in range(2):
            wdescs = [
                pltpu.async_copy(
                    wall_hbm.at[pl.ds((wid * H + h) * EPT
                                      + cc * ECH + half * 512, 512)],
                    wt_v.at[h], sg0)
                for h in range(H)]
            for d in wdescs:
                d.wait()

            def bhead(h, carry2):
                def bgrp(i, carry3):
                    w = wt_v[h, pl.ds(i * 16, 16)]
                    plsc.store_scatter(w16_v, [(i * 256 + h) + iota16 * 16], w)
                    return carry3
                lax.fori_loop(0, 512 // 16, bgrp, 0)
                return carry2
            lax.fori_loop(0, H, bhead, 0)

            pend = None
            sc_pend = [None, None]
            for u in range(8):
                pa = u % 2
                if u == 0:
                    d_g = gstart(0, half, gbufs[0], gsems[0])
                else:
                    d_g = pend
                if u < 7:
                    pend = gstart(u + 1, half, gbufs[(u + 1) % 2],
                                  gsems[(u + 1) % 2])
                rbuf = rbufs[pa]
                if sc_pend[pa] is not None:
                    sc_pend[pa].wait()
                d_g.wait()
                lax.fori_loop(0, 64, make_edge_fn(rbuf, gbufs[pa], u * 64), 0)
                sc_pend[pa] = pltpu.async_copy(
                    rbuf, out_sp.at[dst_v.at[half * 8 + u]],
                    rsems[pa], add=True)
            sc_pend[0].wait()
            sc_pend[1].wait()
        return carry
    lax.fori_loop(0, CH, chunk, 0)

    plsc.subcore_barrier()
    pltpu.sync_copy(out_sp.at[pl.ds(s * NPT, NPT)], outp_hbm.at[c, s])


def kernel(x, edge_index, W, a_src, a_dst):
    wcat = jnp.transpose(W, (1, 0, 2)).reshape(IN_F, FC)
    blk = jnp.repeat(jnp.arange(H), OUT_F)
    ams = jnp.where(blk[:, None] == jnp.arange(H)[None, :],
                    a_src.reshape(FC)[:, None], 0.0)
    amd = jnp.where(blk[:, None] == jnp.arange(H)[None, :],
                    a_dst.reshape(FC)[:, None], 0.0)
    sel = jnp.where(jnp.arange(H)[:, None] == (jnp.arange(FC)[None, :] // 16),
                    1.0, 0.0)

    xpad = jnp.pad(x, ((0, NPAD - N), (0, 0)))

    hcat, alphas, alphad = pl.pallas_call(
        _tc_proj_body,
        grid=(GRID1,),
        in_specs=[
            pl.BlockSpec((ROWB, IN_F), lambda i: (i, 0)),
            pl.BlockSpec((IN_F, FC), lambda i: (0, 0)),
            pl.BlockSpec((IN_F, H), lambda i: (0, 0)),
            pl.BlockSpec((IN_F, H), lambda i: (0, 0)),
        ],
        out_specs=[
            pl.BlockSpec((ROWB, FC), lambda i: (i, 0)),
            pl.BlockSpec((H, ROWB), lambda i: (0, i)),
            pl.BlockSpec((H, ROWB), lambda i: (0, i)),
        ],
        out_shape=[
            jax.ShapeDtypeStruct((NPAD, FC), jnp.float32),
            jax.ShapeDtypeStruct((H, NPAD), jnp.float32),
            jax.ShapeDtypeStruct((H, NPAD), jnp.float32),
        ],
    )(xpad, wcat, ams, amd)

    padi = (N + jnp.arange(EPAD - E, dtype=jnp.int32) % (NPAD - N))
    srcf = jnp.concatenate([edge_index[0], padi])
    dstf = jnp.concatenate([edge_index[1], padi])
    src3d = srcf.reshape(NW, NB, BW)
    dst3d = dstf.reshape(NW, 2 * NB, 64)

    dpart, wall = _sc_logits(alphas.reshape(H * NPAD),
                             alphad.reshape(H * NPAD), srcf, dstf)
    outp = _sc_scatter(src3d, dst3d, hcat, wall)

    dpart = dpart.reshape(NW, H, NPAD)
    parts = outp.reshape(NC, NPAD, FC)

    out = pl.pallas_call(
        _tc_final_body,
        grid=(GRID1,),
        in_specs=[
            pl.BlockSpec((NW, H, ROWB), lambda i: (0, 0, i)),
            pl.BlockSpec((H, FC), lambda i: (0, 0)),
            pl.BlockSpec((ROWB, FC), lambda i: (i, 0)),
            pl.BlockSpec((ROWB, FC), lambda i: (i, 0)),
            pl.BlockSpec((ROWB, FC), lambda i: (i, 0)),
        ],
        out_specs=pl.BlockSpec((ROWB, FC), lambda i: (i, 0)),
        out_shape=jax.ShapeDtypeStruct((NPAD, FC), jnp.float32),
    )(dpart, sel, parts[0], parts[1], xpad)
    return out[:N]
